# Initial kernel scaffold; baseline (speedup 1.0000x reference)
import jax
import jax.numpy as jnp
from jax.experimental import pallas as pl


def kernel(x, xyz, category, b0_w0, b0_g0, b0_b0, b0_w1, b0_g1, b0_b1, b1_w0, b1_g0, b1_b0, b1_w1, b1_g1, b1_b1, b2_w0, b2_g0, b2_b0, mlp1_w, mlp1_g, mlp1_b, cat_emb, m2_w0, m2_g0, m2_b0, m2_w1, m2_g1, m2_b1, m2_w2, m2_bias2):
    raise NotImplementedError("write your pallas kernel here")


# speedup vs baseline: 7.8505x; 7.8505x over previous
import functools

import jax
import jax.numpy as jnp
from jax import lax
from jax.experimental import pallas as pl
from jax.experimental.pallas import tpu as pltpu
from jax.experimental.pallas import tpu_sc as plsc

_K = 20


def _knn_body(f_ref, out_ref, *, n, k, blk):
    b = pl.program_id(0)
    i = pl.program_id(1)
    f = f_ref[0]
    rows = f_ref[0, pl.ds(i * blk, blk), :]
    prod = lax.dot_general(rows, f, (((1,), (1,)), ((), ())),
                           preferred_element_type=jnp.float32)
    rsq = jnp.sum(rows * rows, axis=1, keepdims=True)
    fsq = jnp.sum(f * f, axis=1)[None, :]
    d = (rsq + (-2.0 * prod)) + fsq
    iota = lax.broadcasted_iota(jnp.int32, (blk, n), 1)
    base = b * n
    inf = jnp.float32(jnp.inf)
    for j in range(k):
        m = jnp.min(d, axis=1, keepdims=True)
        ii = jnp.min(jnp.where(d == m, iota, n), axis=1)
        out_ref[0, j, :] = ii + base
        d = jnp.where(iota == ii[:, None], inf, d)


def _knn(feat, k=_K, blk=256):
    b, n, c = feat.shape
    return pl.pallas_call(
        functools.partial(_knn_body, n=n, k=k, blk=blk),
        grid=(b, n // blk),
        in_specs=[pl.BlockSpec((1, n, c), lambda bb, ii: (bb, 0, 0))],
        out_specs=pl.BlockSpec((1, k, blk), lambda bb, ii: (bb, 0, ii)),
        out_shape=jax.ShapeDtypeStruct((b, k, n), jnp.int32),
    )(feat)


_NC = 2
_NS = 16
_NW = _NC * _NS
_CH = 128


def _sc_gather(table, idx):
    e = idx.shape[0]
    d = table.shape[1]
    per_w = e // _NW
    nch = per_w // _CH
    idx3 = idx.reshape(_NW, nch, _CH)
    mesh = plsc.VectorSubcoreMesh(core_axis_name="c", subcore_axis_name="s")

    @functools.partial(
        pl.kernel,
        mesh=mesh,
        compiler_params=pltpu.CompilerParams(use_tc_tiling_on_sc=False),
        out_type=jax.ShapeDtypeStruct((e, d), jnp.float32),
        scratch_types=[
            pltpu.VMEM((nch, _CH), jnp.int32),
            pltpu.VMEM((_CH, d), jnp.float32),
            pltpu.SemaphoreType.DMA,
        ],
    )
    def gk(table_hbm, idx_hbm, out_hbm, idx_v, rows_v, sem):
        wid = lax.axis_index("s") * _NC + lax.axis_index("c")
        pltpu.sync_copy(idx_hbm.at[wid], idx_v)
        base = wid * per_w

        def body(g, carry):
            pltpu.async_copy(table_hbm.at[idx_v.at[g]], rows_v, sem).wait()
            pltpu.sync_copy(rows_v, out_hbm.at[pl.ds(base + g * _CH, _CH)])
            return carry

        lax.fori_loop(0, nch, body, 0)

    return gk(table, idx3)


def _ln_gelu(h, gam, bet):
    mu = jnp.mean(h, axis=1, keepdims=True)
    v = jnp.mean((h - mu) ** 2, axis=1, keepdims=True)
    h = (h - mu) / jnp.sqrt(v + 1e-5) * gam + bet
    return h * 0.5 * (1.0 + lax.erf(h / jnp.sqrt(jnp.float32(2.0))))


def _edge_body(*refs, k, nlayers):
    g_ref, xc_ref = refs[0], refs[1]
    wr = refs[2:2 + 3 * nlayers]
    out_ref = refs[2 + 3 * nlayers]
    xc = xc_ref[...]
    acc = None
    for j in range(k):
        h = jnp.concatenate([g_ref[j] - xc, xc], axis=1)
        for li in range(nlayers):
            w, gam, bet = wr[3 * li][...], wr[3 * li + 1][...], wr[3 * li + 2][...]
            h = lax.dot_general(h, w, (((1,), (1,)), ((), ())),
                                preferred_element_type=jnp.float32)
            h = _ln_gelu(h, gam, bet)
        acc = h if acc is None else jnp.maximum(acc, h)
    out_ref[...] = acc


def _edgeconv(g, xc, layers, k=_K, blk=256):
    m, d = xc.shape
    flat = [a for layer in layers for a in layer]
    in_specs = [
        pl.BlockSpec((k, blk, d), lambda i: (0, i, 0)),
        pl.BlockSpec((blk, d), lambda i: (i, 0)),
    ] + [pl.BlockSpec(a.shape, lambda i, nd=a.ndim: (0,) * nd) for a in flat]
    return pl.pallas_call(
        functools.partial(_edge_body, k=k, nlayers=len(layers)),
        grid=(m // blk,),
        in_specs=in_specs,
        out_specs=pl.BlockSpec((blk, 64), lambda i: (i, 0)),
        out_shape=jax.ShapeDtypeStruct((m, 64), jnp.float32),
    )(g, xc, *flat)


def _head1_body(x1_ref, x2_ref, x3_ref, wa_ref, wb_ref, wc_ref, gam_ref,
                bet_ref, out_ref, *, nblk):
    i = pl.program_id(1)
    dn = (((1,), (1,)), ((), ()))
    h = (lax.dot_general(x1_ref[...], wa_ref[...], dn,
                         preferred_element_type=jnp.float32)
         + lax.dot_general(x2_ref[...], wb_ref[...], dn,
                           preferred_element_type=jnp.float32)
         + lax.dot_general(x3_ref[...], wc_ref[...], dn,
                           preferred_element_type=jnp.float32))
    h = _ln_gelu(h, gam_ref[...], bet_ref[...])
    m = jnp.max(h, axis=0, keepdims=True)

    @pl.when(i == 0)
    def _():
        out_ref[0] = m

    @pl.when(i > 0)
    def _():
        out_ref[0] = jnp.maximum(out_ref[0], m)


def _head1(x1, x2, x3, wa, wb, wc, gam, bet, b, n, blk=512):
    nblk = n // blk
    xspec = pl.BlockSpec((blk, 64), lambda bb, ii: (bb * nblk + ii, 0))
    wspec = pl.BlockSpec((1024, 64), lambda bb, ii: (0, 0))
    vspec = pl.BlockSpec((1, 1024), lambda bb, ii: (0, 0))
    return pl.pallas_call(
        functools.partial(_head1_body, nblk=nblk),
        grid=(b, nblk),
        in_specs=[xspec, xspec, xspec, wspec, wspec, wspec, vspec, vspec],
        out_specs=pl.BlockSpec((1, 1, 1024), lambda bb, ii: (bb, 0, 0)),
        out_shape=jax.ShapeDtypeStruct((b, 1, 1024), jnp.float32),
    )(x1, x2, x3, wa, wb, wc, gam, bet)


def _head2_body(cat_ref, gmax_ref, x1_ref, x2_ref, x3_ref, emb_ref,
                w0g_ref, w0c_ref, w0x1_ref, w0x2_ref, w0x3_ref, g0_ref, b0_ref,
                w1_ref, g1_ref, b1_ref, w2_ref, b2_ref, out_ref):
    dn = (((1,), (1,)), ((), ()))

    def mm(a, b):
        return lax.dot_general(a, b, dn, preferred_element_type=jnp.float32)

    cid = cat_ref[0, 0, 0]
    cvec = emb_ref[pl.ds(cid, 1), :]
    const = mm(gmax_ref[0], w0g_ref[...]) + mm(cvec, w0c_ref[...])
    h = (mm(x1_ref[...], w0x1_ref[...]) + mm(x2_ref[...], w0x2_ref[...])
         + mm(x3_ref[...], w0x3_ref[...]) + const)
    h = _ln_gelu(h, g0_ref[...], b0_ref[...])
    h = _ln_gelu(mm(h, w1_ref[...]), g1_ref[...], b1_ref[...])
    out_ref[...] = mm(h, w2_ref[...]) + b2_ref[...]


def _head2(category, gmax, x1, x2, x3, emb, w0g, w0c, w0x1, w0x2, w0x3,
           g0, b0, w1, g1, b1, w2, b2, b, n, blk=512):
    nblk = n // blk
    xspec = pl.BlockSpec((blk, 64), lambda bb, ii: (bb * nblk + ii, 0))

    def fullspec(a):
        return pl.BlockSpec(a.shape, lambda bb, ii, nd=a.ndim: (0,) * nd)

    return pl.pallas_call(
        _head2_body,
        grid=(b, nblk),
        in_specs=[
            pl.BlockSpec((1, 1, 1), lambda bb, ii: (bb, 0, 0),
                         memory_space=pltpu.SMEM),
            pl.BlockSpec((1, 1, 1024), lambda bb, ii: (bb, 0, 0)),
            xspec, xspec, xspec,
            fullspec(emb), fullspec(w0g), fullspec(w0c), fullspec(w0x1),
            fullspec(w0x2), fullspec(w0x3), fullspec(g0), fullspec(b0),
            fullspec(w1), fullspec(g1), fullspec(b1), fullspec(w2),
            fullspec(b2),
        ],
        out_specs=pl.BlockSpec((blk, 50), lambda bb, ii: (bb * nblk + ii, 0)),
        out_shape=jax.ShapeDtypeStruct((b * n, 50), jnp.float32),
    )(category, gmax, x1, x2, x3, emb, w0g, w0c, w0x1, w0x2, w0x3,
      g0, b0, w1, g1, b1, w2, b2)


def kernel(x, xyz, category, b0_w0, b0_g0, b0_b0, b0_w1, b0_g1, b0_b1,
           b1_w0, b1_g0, b1_b0, b1_w1, b1_g1, b1_b1, b2_w0, b2_g0, b2_b0,
           mlp1_w, mlp1_g, mlp1_b, cat_emb, m2_w0, m2_g0, m2_b0,
           m2_w1, m2_g1, m2_b1, m2_w2, m2_bias2):
    b, n, _ = x.shape
    m = b * n

    def row(v):
        return v.reshape(1, -1).astype(jnp.float32)

    xyz8 = jnp.pad(xyz, ((0, 0), (0, 0), (0, 5)))
    idx1 = _knn(xyz8)
    x16 = jnp.pad(x, ((0, 0), (0, 0), (0, 13))).reshape(m, 16)
    g1 = _sc_gather(x16, idx1.transpose(1, 0, 2).reshape(-1))
    w0p = (jnp.zeros((64, 32), jnp.float32)
           .at[:, 0:3].set(b0_w0[:, 0:3]).at[:, 16:19].set(b0_w0[:, 3:6]))
    x1 = _edgeconv(g1.reshape(_K, m, 16), x16,
                   [(w0p, row(b0_g0), row(b0_b0)),
                    (b0_w1, row(b0_g1), row(b0_b1))])

    idx2 = _knn(x1.reshape(b, n, 64))
    g2 = _sc_gather(x1, idx2.transpose(1, 0, 2).reshape(-1))
    x2 = _edgeconv(g2.reshape(_K, m, 64), x1,
                   [(b1_w0, row(b1_g0), row(b1_b0)),
                    (b1_w1, row(b1_g1), row(b1_b1))])

    idx3 = _knn(x2.reshape(b, n, 64))
    g3 = _sc_gather(x2, idx3.transpose(1, 0, 2).reshape(-1))
    x3 = _edgeconv(g3.reshape(_K, m, 64), x2,
                   [(b2_w0, row(b2_g0), row(b2_b0))])

    gmax = _head1(x1, x2, x3, mlp1_w[:, 0:64], mlp1_w[:, 64:128],
                  mlp1_w[:, 128:192], row(mlp1_g), row(mlp1_b), b, n)
    out = _head2(category.reshape(b, 1, 1).astype(jnp.int32), gmax, x1, x2, x3,
                 cat_emb, m2_w0[:, 0:1024], m2_w0[:, 1024:1088],
                 m2_w0[:, 1088:1152], m2_w0[:, 1152:1216], m2_w0[:, 1216:1280],
                 row(m2_g0), row(m2_b0), m2_w1, row(m2_g1), row(m2_b1),
                 m2_w2, row(m2_bias2), b, n)
    return out.reshape(b, n, 50)


# speedup vs baseline: 7.8843x; 1.0043x over previous
import functools

import jax
import jax.numpy as jnp
from jax import lax
from jax.experimental import pallas as pl
from jax.experimental.pallas import tpu as pltpu
from jax.experimental.pallas import tpu_sc as plsc

_K = 20


def _knn_body(f_ref, out_ref, *, n, k, blk):
    b = pl.program_id(0)
    i = pl.program_id(1)
    f = f_ref[0]
    rows = f_ref[0, pl.ds(i * blk, blk), :]
    prod = lax.dot_general(rows, f, (((1,), (1,)), ((), ())),
                           preferred_element_type=jnp.float32)
    rsq = jnp.sum(rows * rows, axis=1, keepdims=True)
    fsq = jnp.sum(f * f, axis=1)[None, :]
    d = (rsq + (-2.0 * prod)) + fsq
    iota = lax.broadcasted_iota(jnp.int32, (blk, n), 1)
    base = b * n
    inf = jnp.float32(jnp.inf)
    for j in range(k):
        m = jnp.min(d, axis=1, keepdims=True)
        ii = jnp.min(jnp.where(d == m, iota, n), axis=1)
        out_ref[0, j, :] = ii + base
        d = jnp.where(iota == ii[:, None], inf, d)


def _knn(feat, k=_K, blk=256):
    b, n, c = feat.shape
    return pl.pallas_call(
        functools.partial(_knn_body, n=n, k=k, blk=blk),
        grid=(b, n // blk),
        in_specs=[pl.BlockSpec((1, n, c), lambda bb, ii: (bb, 0, 0))],
        out_specs=pl.BlockSpec((1, k, blk), lambda bb, ii: (bb, 0, ii)),
        out_shape=jax.ShapeDtypeStruct((b, k, n), jnp.int32),
    )(feat)


_NC = 2
_NS = 16
_NW = _NC * _NS
_CH = 128


def _sc_gather(table, idx):
    e = idx.shape[0]
    d = table.shape[1]
    per_w = e // _NW
    nch = per_w // _CH
    idx3 = idx.reshape(_NW, nch, _CH)
    mesh = plsc.VectorSubcoreMesh(core_axis_name="c", subcore_axis_name="s")

    @functools.partial(
        pl.kernel,
        mesh=mesh,
        compiler_params=pltpu.CompilerParams(use_tc_tiling_on_sc=False),
        out_type=jax.ShapeDtypeStruct((e, d), jnp.float32),
        scratch_types=[
            pltpu.VMEM((nch, _CH), jnp.int32),
            pltpu.VMEM((_CH, d), jnp.float32),
            pltpu.VMEM((_CH, d), jnp.float32),
            pltpu.SemaphoreType.DMA,
            pltpu.SemaphoreType.DMA,
        ],
    )
    def gk(table_hbm, idx_hbm, out_hbm, idx_v, rows0_v, rows1_v, sem0, sem1):
        wid = lax.axis_index("s") * _NC + lax.axis_index("c")
        pltpu.sync_copy(idx_hbm.at[wid], idx_v)
        base = wid * per_w
        def body(g, carry):
            pltpu.async_copy(table_hbm.at[idx_v.at[g]], rows0_v, sem0).wait()
            pltpu.sync_copy(rows0_v, out_hbm.at[pl.ds(base + g * _CH, _CH)])
            return carry

        lax.fori_loop(0, nch, body, 0)

    return gk(table, idx3)


def _ln_gelu(h, gam, bet):
    mu = jnp.mean(h, axis=1, keepdims=True)
    v = jnp.mean((h - mu) ** 2, axis=1, keepdims=True)
    h = (h - mu) / jnp.sqrt(v + 1e-5) * gam + bet
    return h * 0.5 * (1.0 + lax.erf(h / jnp.sqrt(jnp.float32(2.0))))


def _pre_body(x_ref, wa_ref, wd_ref, y_ref, z_ref):
    dn = (((1,), (1,)), ((), ()))
    x = x_ref[...]
    y_ref[...] = lax.dot_general(x, wa_ref[...], dn,
                                 preferred_element_type=jnp.float32)
    z_ref[...] = lax.dot_general(x, wd_ref[...], dn,
                                 preferred_element_type=jnp.float32)


def _pretrans(x, wa, wd, blk=1024):
    m, d = x.shape
    ospec = pl.BlockSpec((blk, 64), lambda i: (i, 0))
    oshape = jax.ShapeDtypeStruct((m, 64), jnp.float32)
    return pl.pallas_call(
        _pre_body,
        grid=(m // blk,),
        in_specs=[
            pl.BlockSpec((blk, d), lambda i: (i, 0)),
            pl.BlockSpec(wa.shape, lambda i: (0, 0)),
            pl.BlockSpec(wd.shape, lambda i: (0, 0)),
        ],
        out_specs=[ospec, ospec],
        out_shape=[oshape, oshape],
    )(x, wa, wd)


def _edge_body(*refs, k, two):
    gy_ref, xc_ref, w0_ref, g0_ref, b0_ref = refs[:5]
    out_ref = refs[-1]
    xc = xc_ref[...]
    w0 = w0_ref[...]
    g0, b0 = g0_ref[...], b0_ref[...]
    if two:
        w1, g1, b1 = refs[5][...], refs[6][...], refs[7][...]
    acc = None
    for j in range(k):
        h = jnp.concatenate([gy_ref[j] - xc, xc], axis=1)
        h = lax.dot_general(h, w0, (((1,), (1,)), ((), ())),
                            preferred_element_type=jnp.float32)
        h = _ln_gelu(h, g0, b0)
        if two:
            h = lax.dot_general(h, w1, (((1,), (1,)), ((), ())),
                                preferred_element_type=jnp.float32)
            h = _ln_gelu(h, g1, b1)
        acc = h if acc is None else jnp.maximum(acc, h)
    out_ref[...] = acc


def _edge3_body(gy_ref, z_ref, g0_ref, b0_ref, out_ref, *, k):
    z = z_ref[...]
    g0, b0 = g0_ref[...], b0_ref[...]
    acc = None
    for j in range(k):
        h = _ln_gelu(gy_ref[j] + z, g0, b0)
        acc = h if acc is None else jnp.maximum(acc, h)
    out_ref[...] = acc


def _edgeconv3(gy, z, g0, b0, k=_K, blk=512):
    m = z.shape[0]
    return pl.pallas_call(
        functools.partial(_edge3_body, k=k),
        grid=(m // blk,),
        in_specs=[
            pl.BlockSpec((k, blk, 64), lambda i: (0, i, 0)),
            pl.BlockSpec((blk, 64), lambda i: (i, 0)),
            pl.BlockSpec(g0.shape, lambda i: (0, 0)),
            pl.BlockSpec(b0.shape, lambda i: (0, 0)),
        ],
        out_specs=pl.BlockSpec((blk, 64), lambda i: (i, 0)),
        out_shape=jax.ShapeDtypeStruct((m, 64), jnp.float32),
    )(gy, z, g0, b0)


def _edgeconv(gy, xc, w0, g0, b0, layer2=None, k=_K, blk=256):
    m, d = xc.shape
    extra = list(layer2) if layer2 is not None else []
    in_specs = [
        pl.BlockSpec((k, blk, d), lambda i: (0, i, 0)),
        pl.BlockSpec((blk, d), lambda i: (i, 0)),
        pl.BlockSpec(w0.shape, lambda i: (0, 0)),
        pl.BlockSpec(g0.shape, lambda i: (0, 0)),
        pl.BlockSpec(b0.shape, lambda i: (0, 0)),
    ] + [pl.BlockSpec(a.shape, lambda i, nd=a.ndim: (0,) * nd) for a in extra]
    return pl.pallas_call(
        functools.partial(_edge_body, k=k, two=layer2 is not None),
        grid=(m // blk,),
        in_specs=in_specs,
        out_specs=pl.BlockSpec((blk, 64), lambda i: (i, 0)),
        out_shape=jax.ShapeDtypeStruct((m, 64), jnp.float32),
    )(gy, xc, w0, g0, b0, *extra)


def _head1_body(x1_ref, x2_ref, x3_ref, wa_ref, wb_ref, wc_ref, gam_ref,
                bet_ref, out_ref, *, nblk):
    i = pl.program_id(1)
    dn = (((1,), (1,)), ((), ()))
    h = (lax.dot_general(x1_ref[...], wa_ref[...], dn,
                         preferred_element_type=jnp.float32)
         + lax.dot_general(x2_ref[...], wb_ref[...], dn,
                           preferred_element_type=jnp.float32)
         + lax.dot_general(x3_ref[...], wc_ref[...], dn,
                           preferred_element_type=jnp.float32))
    h = _ln_gelu(h, gam_ref[...], bet_ref[...])
    m = jnp.max(h, axis=0, keepdims=True)

    @pl.when(i == 0)
    def _():
        out_ref[0] = m

    @pl.when(i > 0)
    def _():
        out_ref[0] = jnp.maximum(out_ref[0], m)


def _head1(x1, x2, x3, wa, wb, wc, gam, bet, b, n, blk=512):
    nblk = n // blk
    xspec = pl.BlockSpec((blk, 64), lambda bb, ii: (bb * nblk + ii, 0))
    wspec = pl.BlockSpec((1024, 64), lambda bb, ii: (0, 0))
    vspec = pl.BlockSpec((1, 1024), lambda bb, ii: (0, 0))
    return pl.pallas_call(
        functools.partial(_head1_body, nblk=nblk),
        grid=(b, nblk),
        in_specs=[xspec, xspec, xspec, wspec, wspec, wspec, vspec, vspec],
        out_specs=pl.BlockSpec((1, 1, 1024), lambda bb, ii: (bb, 0, 0)),
        out_shape=jax.ShapeDtypeStruct((b, 1, 1024), jnp.float32),
    )(x1, x2, x3, wa, wb, wc, gam, bet)


def _head2_body(cat_ref, gmax_ref, x1_ref, x2_ref, x3_ref, emb_ref,
                w0g_ref, w0c_ref, w0x1_ref, w0x2_ref, w0x3_ref, g0_ref, b0_ref,
                w1_ref, g1_ref, b1_ref, w2_ref, b2_ref, out_ref):
    dn = (((1,), (1,)), ((), ()))

    def mm(a, b):
        return lax.dot_general(a, b, dn, preferred_element_type=jnp.float32)

    cid = cat_ref[0, 0, 0]
    cvec = emb_ref[pl.ds(cid, 1), :]
    const = mm(gmax_ref[0], w0g_ref[...]) + mm(cvec, w0c_ref[...])
    h = (mm(x1_ref[...], w0x1_ref[...]) + mm(x2_ref[...], w0x2_ref[...])
         + mm(x3_ref[...], w0x3_ref[...]) + const)
    h = _ln_gelu(h, g0_ref[...], b0_ref[...])
    h = _ln_gelu(mm(h, w1_ref[...]), g1_ref[...], b1_ref[...])
    out_ref[...] = mm(h, w2_ref[...]) + b2_ref[...]


def _head2(category, gmax, x1, x2, x3, emb, w0g, w0c, w0x1, w0x2, w0x3,
           g0, b0, w1, g1, b1, w2, b2, b, n, blk=512):
    nblk = n // blk
    xspec = pl.BlockSpec((blk, 64), lambda bb, ii: (bb * nblk + ii, 0))

    def fullspec(a):
        return pl.BlockSpec(a.shape, lambda bb, ii, nd=a.ndim: (0,) * nd)

    return pl.pallas_call(
        _head2_body,
        grid=(b, nblk),
        in_specs=[
            pl.BlockSpec((1, 1, 1), lambda bb, ii: (bb, 0, 0),
                         memory_space=pltpu.SMEM),
            pl.BlockSpec((1, 1, 1024), lambda bb, ii: (bb, 0, 0)),
            xspec, xspec, xspec,
            fullspec(emb), fullspec(w0g), fullspec(w0c), fullspec(w0x1),
            fullspec(w0x2), fullspec(w0x3), fullspec(g0), fullspec(b0),
            fullspec(w1), fullspec(g1), fullspec(b1), fullspec(w2),
            fullspec(b2),
        ],
        out_specs=pl.BlockSpec((blk, 50), lambda bb, ii: (bb * nblk + ii, 0)),
        out_shape=jax.ShapeDtypeStruct((b * n, 50), jnp.float32),
    )(category, gmax, x1, x2, x3, emb, w0g, w0c, w0x1, w0x2, w0x3,
      g0, b0, w1, g1, b1, w2, b2)


def kernel(x, xyz, category, b0_w0, b0_g0, b0_b0, b0_w1, b0_g1, b0_b1,
           b1_w0, b1_g0, b1_b0, b1_w1, b1_g1, b1_b1, b2_w0, b2_g0, b2_b0,
           mlp1_w, mlp1_g, mlp1_b, cat_emb, m2_w0, m2_g0, m2_b0,
           m2_w1, m2_g1, m2_b1, m2_w2, m2_bias2):
    b, n, _ = x.shape
    m = b * n

    def row(v):
        return v.reshape(1, -1).astype(jnp.float32)

    xyz8 = jnp.pad(xyz, ((0, 0), (0, 0), (0, 5)))
    idx1 = _knn(xyz8)
    x16 = jnp.pad(x, ((0, 0), (0, 0), (0, 13))).reshape(m, 16)
    w0p = (jnp.zeros((64, 32), jnp.float32)
           .at[:, 0:3].set(b0_w0[:, 0:3]).at[:, 16:19].set(b0_w0[:, 3:6]))
    g1 = _sc_gather(x16, idx1.transpose(1, 0, 2).reshape(-1))
    x1 = _edgeconv(g1.reshape(_K, m, 16), x16, w0p, row(b0_g0), row(b0_b0),
                   layer2=(b0_w1, row(b0_g1), row(b0_b1)))

    idx2 = _knn(x1.reshape(b, n, 64))
    g2 = _sc_gather(x1, idx2.transpose(1, 0, 2).reshape(-1))
    x2 = _edgeconv(g2.reshape(_K, m, 64), x1, b1_w0, row(b1_g0), row(b1_b0),
                   layer2=(b1_w1, row(b1_g1), row(b1_b1)))

    idx3 = _knn(x2.reshape(b, n, 64))
    y3, z3 = _pretrans(x2, b2_w0[:, 0:64], b2_w0[:, 64:128] - b2_w0[:, 0:64])
    g3 = _sc_gather(y3, idx3.transpose(1, 0, 2).reshape(-1))
    x3 = _edgeconv3(g3.reshape(_K, m, 64), z3, row(b2_g0), row(b2_b0))

    gmax = _head1(x1, x2, x3, mlp1_w[:, 0:64], mlp1_w[:, 64:128],
                  mlp1_w[:, 128:192], row(mlp1_g), row(mlp1_b), b, n)
    out = _head2(category.reshape(b, 1, 1).astype(jnp.int32), gmax, x1, x2, x3,
                 cat_emb, m2_w0[:, 0:1024], m2_w0[:, 1024:1088],
                 m2_w0[:, 1088:1152], m2_w0[:, 1152:1216], m2_w0[:, 1216:1280],
                 row(m2_g0), row(m2_b0), m2_w1, row(m2_g1), row(m2_b1),
                 m2_w2, row(m2_bias2), b, n)
    return out.reshape(b, n, 50)


# speedup vs baseline: 8.3720x; 1.0619x over previous
import functools

import jax
import jax.numpy as jnp
from jax import lax
from jax.experimental import pallas as pl
from jax.experimental.pallas import tpu as pltpu
from jax.experimental.pallas import tpu_sc as plsc

_K = 20


def _knn_body(f_ref, out_ref, *, n, k, blk):
    b = pl.program_id(0)
    i = pl.program_id(1)
    f = f_ref[0]
    rows = f_ref[0, pl.ds(i * blk, blk), :]
    prod = lax.dot_general(rows, f, (((1,), (1,)), ((), ())),
                           preferred_element_type=jnp.float32)
    rsq = jnp.sum(rows * rows, axis=1, keepdims=True)
    fsq = jnp.sum(f * f, axis=1)[None, :]
    d = (rsq + (-2.0 * prod)) + fsq
    iota = lax.broadcasted_iota(jnp.int32, (blk, n), 1)
    base = b * n
    inf = jnp.float32(jnp.inf)
    for j in range(k):
        m = jnp.min(d, axis=1, keepdims=True)
        t = jnp.where(d == m, iota, n)
        ii = jnp.min(t, axis=1)
        out_ref[0, j, :] = ii + base
        d = jnp.where(t == ii[:, None], inf, d)


def _knn(feat, k=_K, blk=256):
    b, n, c = feat.shape
    return pl.pallas_call(
        functools.partial(_knn_body, n=n, k=k, blk=blk),
        grid=(b, n // blk),
        in_specs=[pl.BlockSpec((1, n, c), lambda bb, ii: (bb, 0, 0))],
        out_specs=pl.BlockSpec((1, k, blk), lambda bb, ii: (bb, 0, ii)),
        out_shape=jax.ShapeDtypeStruct((b, k, n), jnp.int32),
    )(feat)


_NC = 2
_NS = 16
_NW = _NC * _NS
_CH = 128


def _sc_gather(table, idx):
    e = idx.shape[0]
    d = table.shape[1]
    per_w = e // _NW
    nch = per_w // _CH
    idx3 = idx.reshape(_NW, nch, _CH)
    mesh = plsc.VectorSubcoreMesh(core_axis_name="c", subcore_axis_name="s")

    @functools.partial(
        pl.kernel,
        mesh=mesh,
        compiler_params=pltpu.CompilerParams(use_tc_tiling_on_sc=False),
        out_type=jax.ShapeDtypeStruct((e, d), jnp.float32),
        scratch_types=[
            pltpu.VMEM((nch, _CH), jnp.int32),
            pltpu.VMEM((_CH, d), jnp.float32),
            pltpu.VMEM((_CH, d), jnp.float32),
            pltpu.SemaphoreType.DMA,
            pltpu.SemaphoreType.DMA,
        ],
    )
    def gk(table_hbm, idx_hbm, out_hbm, idx_v, rows0_v, rows1_v, sem0, sem1):
        wid = lax.axis_index("s") * _NC + lax.axis_index("c")
        pltpu.sync_copy(idx_hbm.at[wid], idx_v)
        base = wid * per_w
        pltpu.async_copy(table_hbm.at[idx_v.at[0]], rows0_v, sem0)

        def step(g, cur, nxt, cur_sem, nxt_sem):
            pltpu.make_async_copy(table_hbm.at[idx_v.at[g]], cur,
                                  cur_sem).wait()

            @pl.when(g + 1 < nch)
            def _():
                pltpu.async_copy(table_hbm.at[idx_v.at[g + 1]], nxt, nxt_sem)

            pltpu.sync_copy(cur, out_hbm.at[pl.ds(base + g * _CH, _CH)])

        def pair(h, carry):
            step(2 * h, rows0_v, rows1_v, sem0, sem1)
            step(2 * h + 1, rows1_v, rows0_v, sem1, sem0)
            return carry

        lax.fori_loop(0, nch // 2, pair, 0)

    return gk(table, idx3)


def _ln_gelu(h, gam, bet):
    mu = jnp.mean(h, axis=1, keepdims=True)
    v = jnp.mean((h - mu) ** 2, axis=1, keepdims=True)
    h = (h - mu) / jnp.sqrt(v + 1e-5) * gam + bet
    return h * 0.5 * (1.0 + lax.erf(h / jnp.sqrt(jnp.float32(2.0))))


def _pre_body(x_ref, wa_ref, wd_ref, y_ref, z_ref):
    dn = (((1,), (1,)), ((), ()))
    x = x_ref[...]
    y_ref[...] = lax.dot_general(x, wa_ref[...], dn,
                                 preferred_element_type=jnp.float32)
    z_ref[...] = lax.dot_general(x, wd_ref[...], dn,
                                 preferred_element_type=jnp.float32)


def _pretrans(x, wa, wd, blk=1024):
    m, d = x.shape
    ospec = pl.BlockSpec((blk, 64), lambda i: (i, 0))
    oshape = jax.ShapeDtypeStruct((m, 64), jnp.float32)
    return pl.pallas_call(
        _pre_body,
        grid=(m // blk,),
        in_specs=[
            pl.BlockSpec((blk, d), lambda i: (i, 0)),
            pl.BlockSpec(wa.shape, lambda i: (0, 0)),
            pl.BlockSpec(wd.shape, lambda i: (0, 0)),
        ],
        out_specs=[ospec, ospec],
        out_shape=[oshape, oshape],
    )(x, wa, wd)


def _edge_body(*refs, k, two):
    gy_ref, xc_ref, w0_ref, g0_ref, b0_ref = refs[:5]
    out_ref = refs[-1]
    xc = xc_ref[...]
    w0 = w0_ref[...]
    g0, b0 = g0_ref[...], b0_ref[...]
    if two:
        w1, g1, b1 = refs[5][...], refs[6][...], refs[7][...]
    acc = None
    for j in range(k):
        h = jnp.concatenate([gy_ref[j] - xc, xc], axis=1)
        h = lax.dot_general(h, w0, (((1,), (1,)), ((), ())),
                            preferred_element_type=jnp.float32)
        h = _ln_gelu(h, g0, b0)
        if two:
            h = lax.dot_general(h, w1, (((1,), (1,)), ((), ())),
                                preferred_element_type=jnp.float32)
            h = _ln_gelu(h, g1, b1)
        acc = h if acc is None else jnp.maximum(acc, h)
    out_ref[...] = acc


def _edge3_body(gy_ref, z_ref, g0_ref, b0_ref, out_ref, *, k):
    z = z_ref[...]
    g0, b0 = g0_ref[...], b0_ref[...]
    acc = None
    for j in range(k):
        h = _ln_gelu(gy_ref[j] + z, g0, b0)
        acc = h if acc is None else jnp.maximum(acc, h)
    out_ref[...] = acc


def _edgeconv3(gy, z, g0, b0, k=_K, blk=512):
    m = z.shape[0]
    return pl.pallas_call(
        functools.partial(_edge3_body, k=k),
        grid=(m // blk,),
        in_specs=[
            pl.BlockSpec((k, blk, 64), lambda i: (0, i, 0)),
            pl.BlockSpec((blk, 64), lambda i: (i, 0)),
            pl.BlockSpec(g0.shape, lambda i: (0, 0)),
            pl.BlockSpec(b0.shape, lambda i: (0, 0)),
        ],
        out_specs=pl.BlockSpec((blk, 64), lambda i: (i, 0)),
        out_shape=jax.ShapeDtypeStruct((m, 64), jnp.float32),
    )(gy, z, g0, b0)


def _edgeconv(gy, xc, w0, g0, b0, layer2=None, k=_K, blk=512):
    m, d = xc.shape
    extra = list(layer2) if layer2 is not None else []
    in_specs = [
        pl.BlockSpec((k, blk, d), lambda i: (0, i, 0)),
        pl.BlockSpec((blk, d), lambda i: (i, 0)),
        pl.BlockSpec(w0.shape, lambda i: (0, 0)),
        pl.BlockSpec(g0.shape, lambda i: (0, 0)),
        pl.BlockSpec(b0.shape, lambda i: (0, 0)),
    ] + [pl.BlockSpec(a.shape, lambda i, nd=a.ndim: (0,) * nd) for a in extra]
    return pl.pallas_call(
        functools.partial(_edge_body, k=k, two=layer2 is not None),
        grid=(m // blk,),
        in_specs=in_specs,
        out_specs=pl.BlockSpec((blk, 64), lambda i: (i, 0)),
        out_shape=jax.ShapeDtypeStruct((m, 64), jnp.float32),
    )(gy, xc, w0, g0, b0, *extra)


def _head1_body(x1_ref, x2_ref, x3_ref, wa_ref, wb_ref, wc_ref, gam_ref,
                bet_ref, out_ref, *, nblk):
    i = pl.program_id(1)
    dn = (((1,), (1,)), ((), ()))
    h = (lax.dot_general(x1_ref[...], wa_ref[...], dn,
                         preferred_element_type=jnp.float32)
         + lax.dot_general(x2_ref[...], wb_ref[...], dn,
                           preferred_element_type=jnp.float32)
         + lax.dot_general(x3_ref[...], wc_ref[...], dn,
                           preferred_element_type=jnp.float32))
    h = _ln_gelu(h, gam_ref[...], bet_ref[...])
    m = jnp.max(h, axis=0, keepdims=True)

    @pl.when(i == 0)
    def _():
        out_ref[0] = m

    @pl.when(i > 0)
    def _():
        out_ref[0] = jnp.maximum(out_ref[0], m)


def _head1(x1, x2, x3, wa, wb, wc, gam, bet, b, n, blk=512):
    nblk = n // blk
    xspec = pl.BlockSpec((blk, 64), lambda bb, ii: (bb * nblk + ii, 0))
    wspec = pl.BlockSpec((1024, 64), lambda bb, ii: (0, 0))
    vspec = pl.BlockSpec((1, 1024), lambda bb, ii: (0, 0))
    return pl.pallas_call(
        functools.partial(_head1_body, nblk=nblk),
        grid=(b, nblk),
        in_specs=[xspec, xspec, xspec, wspec, wspec, wspec, vspec, vspec],
        out_specs=pl.BlockSpec((1, 1, 1024), lambda bb, ii: (bb, 0, 0)),
        out_shape=jax.ShapeDtypeStruct((b, 1, 1024), jnp.float32),
    )(x1, x2, x3, wa, wb, wc, gam, bet)


def _head2_body(cat_ref, gmax_ref, x1_ref, x2_ref, x3_ref, emb_ref,
                w0g_ref, w0c_ref, w0x1_ref, w0x2_ref, w0x3_ref, g0_ref, b0_ref,
                w1_ref, g1_ref, b1_ref, w2_ref, b2_ref, out_ref):
    dn = (((1,), (1,)), ((), ()))

    def mm(a, b):
        return lax.dot_general(a, b, dn, preferred_element_type=jnp.float32)

    cid = cat_ref[0, 0, 0]
    cvec = emb_ref[pl.ds(cid, 1), :]
    const = mm(gmax_ref[0], w0g_ref[...]) + mm(cvec, w0c_ref[...])
    h = (mm(x1_ref[...], w0x1_ref[...]) + mm(x2_ref[...], w0x2_ref[...])
         + mm(x3_ref[...], w0x3_ref[...]) + const)
    h = _ln_gelu(h, g0_ref[...], b0_ref[...])
    h = _ln_gelu(mm(h, w1_ref[...]), g1_ref[...], b1_ref[...])
    out_ref[...] = mm(h, w2_ref[...]) + b2_ref[...]


def _head2(category, gmax, x1, x2, x3, emb, w0g, w0c, w0x1, w0x2, w0x3,
           g0, b0, w1, g1, b1, w2, b2, b, n, blk=512):
    nblk = n // blk
    xspec = pl.BlockSpec((blk, 64), lambda bb, ii: (bb * nblk + ii, 0))

    def fullspec(a):
        return pl.BlockSpec(a.shape, lambda bb, ii, nd=a.ndim: (0,) * nd)

    return pl.pallas_call(
        _head2_body,
        grid=(b, nblk),
        in_specs=[
            pl.BlockSpec((1, 1, 1), lambda bb, ii: (bb, 0, 0),
                         memory_space=pltpu.SMEM),
            pl.BlockSpec((1, 1, 1024), lambda bb, ii: (bb, 0, 0)),
            xspec, xspec, xspec,
            fullspec(emb), fullspec(w0g), fullspec(w0c), fullspec(w0x1),
            fullspec(w0x2), fullspec(w0x3), fullspec(g0), fullspec(b0),
            fullspec(w1), fullspec(g1), fullspec(b1), fullspec(w2),
            fullspec(b2),
        ],
        out_specs=pl.BlockSpec((blk, 50), lambda bb, ii: (bb * nblk + ii, 0)),
        out_shape=jax.ShapeDtypeStruct((b * n, 50), jnp.float32),
    )(category, gmax, x1, x2, x3, emb, w0g, w0c, w0x1, w0x2, w0x3,
      g0, b0, w1, g1, b1, w2, b2)


def kernel(x, xyz, category, b0_w0, b0_g0, b0_b0, b0_w1, b0_g1, b0_b1,
           b1_w0, b1_g0, b1_b0, b1_w1, b1_g1, b1_b1, b2_w0, b2_g0, b2_b0,
           mlp1_w, mlp1_g, mlp1_b, cat_emb, m2_w0, m2_g0, m2_b0,
           m2_w1, m2_g1, m2_b1, m2_w2, m2_bias2):
    b, n, _ = x.shape
    m = b * n

    def row(v):
        return v.reshape(1, -1).astype(jnp.float32)

    xyz8 = jnp.pad(xyz, ((0, 0), (0, 0), (0, 5)))
    idx1 = _knn(xyz8)
    x16 = jnp.pad(x, ((0, 0), (0, 0), (0, 13))).reshape(m, 16)
    w0p = (jnp.zeros((64, 32), jnp.float32)
           .at[:, 0:3].set(b0_w0[:, 0:3]).at[:, 16:19].set(b0_w0[:, 3:6]))
    g1 = _sc_gather(x16, idx1.transpose(1, 0, 2).reshape(-1))
    x1 = _edgeconv(g1.reshape(_K, m, 16), x16, w0p, row(b0_g0), row(b0_b0),
                   layer2=(b0_w1, row(b0_g1), row(b0_b1)))

    idx2 = _knn(x1.reshape(b, n, 64))
    g2 = _sc_gather(x1, idx2.transpose(1, 0, 2).reshape(-1))
    x2 = _edgeconv(g2.reshape(_K, m, 64), x1, b1_w0, row(b1_g0), row(b1_b0),
                   layer2=(b1_w1, row(b1_g1), row(b1_b1)))

    idx3 = _knn(x2.reshape(b, n, 64))
    y3, z3 = _pretrans(x2, b2_w0[:, 0:64], b2_w0[:, 64:128] - b2_w0[:, 0:64])
    g3 = _sc_gather(y3, idx3.transpose(1, 0, 2).reshape(-1))
    x3 = _edgeconv3(g3.reshape(_K, m, 64), z3, row(b2_g0), row(b2_b0))

    gmax = _head1(x1, x2, x3, mlp1_w[:, 0:64], mlp1_w[:, 64:128],
                  mlp1_w[:, 128:192], row(mlp1_g), row(mlp1_b), b, n)
    out = _head2(category.reshape(b, 1, 1).astype(jnp.int32), gmax, x1, x2, x3,
                 cat_emb, m2_w0[:, 0:1024], m2_w0[:, 1024:1088],
                 m2_w0[:, 1088:1152], m2_w0[:, 1152:1216], m2_w0[:, 1216:1280],
                 row(m2_g0), row(m2_b0), m2_w1, row(m2_g1), row(m2_b1),
                 m2_w2, row(m2_bias2), b, n)
    return out.reshape(b, n, 50)


# speedup vs baseline: 8.5115x; 1.0167x over previous
import functools

import jax
import jax.numpy as jnp
from jax import lax
from jax.experimental import pallas as pl
from jax.experimental.pallas import tpu as pltpu
from jax.experimental.pallas import tpu_sc as plsc

_K = 20


def _knn_body(f_ref, out_ref, *, n, k, blk):
    b = pl.program_id(0)
    i = pl.program_id(1)
    f = f_ref[0]
    rows = f_ref[0, pl.ds(i * blk, blk), :]
    prod = lax.dot_general(rows, f, (((1,), (1,)), ((), ())),
                           preferred_element_type=jnp.float32)
    rsq = jnp.sum(rows * rows, axis=1, keepdims=True)
    fsq = jnp.sum(f * f, axis=1)[None, :]
    d = (rsq + (-2.0 * prod)) + fsq
    iota = lax.broadcasted_iota(jnp.int32, (blk, n), 1)
    base = b * n
    inf = jnp.float32(jnp.inf)
    for j in range(k):
        m = jnp.min(d, axis=1, keepdims=True)
        ii = jnp.min(jnp.where(d == m, iota, n), axis=1)
        out_ref[0, j, :] = ii + base
        d = jnp.where(iota == ii[:, None], inf, d)


def _knn(feat, k=_K, blk=256):
    b, n, c = feat.shape
    return pl.pallas_call(
        functools.partial(_knn_body, n=n, k=k, blk=blk),
        grid=(b, n // blk),
        in_specs=[pl.BlockSpec((1, n, c), lambda bb, ii: (bb, 0, 0))],
        out_specs=pl.BlockSpec((1, k, blk), lambda bb, ii: (bb, 0, ii)),
        out_shape=jax.ShapeDtypeStruct((b, k, n), jnp.int32),
    )(feat)


_NC = 2
_NS = 16
_NW = _NC * _NS
_CH = 128


def _sc_gather(table, idx):
    e = idx.shape[0]
    d = table.shape[1]
    per_w = e // _NW
    nch = per_w // _CH
    idx3 = idx.reshape(_NW, nch, _CH)
    mesh = plsc.VectorSubcoreMesh(core_axis_name="c", subcore_axis_name="s")

    @functools.partial(
        pl.kernel,
        mesh=mesh,
        compiler_params=pltpu.CompilerParams(use_tc_tiling_on_sc=False),
        out_type=jax.ShapeDtypeStruct((e, d), jnp.float32),
        scratch_types=[
            pltpu.VMEM((nch, _CH), jnp.int32),
            pltpu.VMEM((_CH, d), jnp.float32),
            pltpu.VMEM((_CH, d), jnp.float32),
            pltpu.SemaphoreType.DMA,
            pltpu.SemaphoreType.DMA,
        ],
    )
    def gk(table_hbm, idx_hbm, out_hbm, idx_v, rows0_v, rows1_v, sem0, sem1):
        wid = lax.axis_index("s") * _NC + lax.axis_index("c")
        pltpu.sync_copy(idx_hbm.at[wid], idx_v)
        base = wid * per_w
        pltpu.async_copy(table_hbm.at[idx_v.at[0]], rows0_v, sem0)

        def step(g, cur, nxt, cur_sem, nxt_sem):
            pltpu.make_async_copy(table_hbm.at[idx_v.at[g]], cur,
                                  cur_sem).wait()

            @pl.when(g + 1 < nch)
            def _():
                pltpu.async_copy(table_hbm.at[idx_v.at[g + 1]], nxt, nxt_sem)

            pltpu.sync_copy(cur, out_hbm.at[pl.ds(base + g * _CH, _CH)])

        def pair(h, carry):
            step(2 * h, rows0_v, rows1_v, sem0, sem1)
            step(2 * h + 1, rows1_v, rows0_v, sem1, sem0)
            return carry

        lax.fori_loop(0, nch // 2, pair, 0)

    return gk(table, idx3)


def _ln_gelu(h, gam, bet):
    mu = jnp.mean(h, axis=1, keepdims=True)
    v = jnp.mean((h - mu) ** 2, axis=1, keepdims=True)
    h = (h - mu) / jnp.sqrt(v + 1e-5) * gam + bet
    return h * 0.5 * (1.0 + lax.erf(h / jnp.sqrt(jnp.float32(2.0))))


def _pre_body(x_ref, wa_ref, wd_ref, y_ref, z_ref):
    dn = (((1,), (1,)), ((), ()))
    x = x_ref[...]
    y_ref[...] = lax.dot_general(x, wa_ref[...], dn,
                                 preferred_element_type=jnp.float32)
    z_ref[...] = lax.dot_general(x, wd_ref[...], dn,
                                 preferred_element_type=jnp.float32)


def _pretrans(x, wa, wd, blk=1024):
    m, d = x.shape
    ospec = pl.BlockSpec((blk, 64), lambda i: (i, 0))
    oshape = jax.ShapeDtypeStruct((m, 64), jnp.float32)
    return pl.pallas_call(
        _pre_body,
        grid=(m // blk,),
        in_specs=[
            pl.BlockSpec((blk, d), lambda i: (i, 0)),
            pl.BlockSpec(wa.shape, lambda i: (0, 0)),
            pl.BlockSpec(wd.shape, lambda i: (0, 0)),
        ],
        out_specs=[ospec, ospec],
        out_shape=[oshape, oshape],
    )(x, wa, wd)


def _edge_body(*refs, k, two):
    gy_ref, xc_ref, w0_ref, g0_ref, b0_ref = refs[:5]
    out_ref = refs[-1]
    xc = xc_ref[...]
    w0 = w0_ref[...]
    g0, b0 = g0_ref[...], b0_ref[...]
    if two:
        w1, g1, b1 = refs[5][...], refs[6][...], refs[7][...]
    acc = None
    for j in range(k):
        h = jnp.concatenate([gy_ref[j] - xc, xc], axis=1)
        h = lax.dot_general(h, w0, (((1,), (1,)), ((), ())),
                            preferred_element_type=jnp.float32)
        h = _ln_gelu(h, g0, b0)
        if two:
            h = lax.dot_general(h, w1, (((1,), (1,)), ((), ())),
                                preferred_element_type=jnp.float32)
            h = _ln_gelu(h, g1, b1)
        acc = h if acc is None else jnp.maximum(acc, h)
    out_ref[...] = acc


def _edge3_body(gy_ref, z_ref, g0_ref, b0_ref, out_ref, *, k):
    z = z_ref[...]
    g0, b0 = g0_ref[...], b0_ref[...]
    acc = None
    for j in range(k):
        h = _ln_gelu(gy_ref[j] + z, g0, b0)
        acc = h if acc is None else jnp.maximum(acc, h)
    out_ref[...] = acc


def _edgeconv3(gy, z, g0, b0, k=_K, blk=1024):
    m = z.shape[0]
    return pl.pallas_call(
        functools.partial(_edge3_body, k=k),
        grid=(m // blk,),
        in_specs=[
            pl.BlockSpec((k, blk, 64), lambda i: (0, i, 0)),
            pl.BlockSpec((blk, 64), lambda i: (i, 0)),
            pl.BlockSpec(g0.shape, lambda i: (0, 0)),
            pl.BlockSpec(b0.shape, lambda i: (0, 0)),
        ],
        out_specs=pl.BlockSpec((blk, 64), lambda i: (i, 0)),
        out_shape=jax.ShapeDtypeStruct((m, 64), jnp.float32),
    )(gy, z, g0, b0)


def _edgeconv(gy, xc, w0, g0, b0, layer2=None, k=_K, blk=1024):
    m, d = xc.shape
    extra = list(layer2) if layer2 is not None else []
    in_specs = [
        pl.BlockSpec((k, blk, d), lambda i: (0, i, 0)),
        pl.BlockSpec((blk, d), lambda i: (i, 0)),
        pl.BlockSpec(w0.shape, lambda i: (0, 0)),
        pl.BlockSpec(g0.shape, lambda i: (0, 0)),
        pl.BlockSpec(b0.shape, lambda i: (0, 0)),
    ] + [pl.BlockSpec(a.shape, lambda i, nd=a.ndim: (0,) * nd) for a in extra]
    return pl.pallas_call(
        functools.partial(_edge_body, k=k, two=layer2 is not None),
        grid=(m // blk,),
        in_specs=in_specs,
        out_specs=pl.BlockSpec((blk, 64), lambda i: (i, 0)),
        out_shape=jax.ShapeDtypeStruct((m, 64), jnp.float32),
    )(gy, xc, w0, g0, b0, *extra)


def _head1_body(x1_ref, x2_ref, x3_ref, wa_ref, wb_ref, wc_ref, gam_ref,
                bet_ref, out_ref, *, nblk):
    i = pl.program_id(1)
    dn = (((1,), (1,)), ((), ()))
    h = (lax.dot_general(x1_ref[...], wa_ref[...], dn,
                         preferred_element_type=jnp.float32)
         + lax.dot_general(x2_ref[...], wb_ref[...], dn,
                           preferred_element_type=jnp.float32)
         + lax.dot_general(x3_ref[...], wc_ref[...], dn,
                           preferred_element_type=jnp.float32))
    h = _ln_gelu(h, gam_ref[...], bet_ref[...])
    m = jnp.max(h, axis=0, keepdims=True)

    @pl.when(i == 0)
    def _():
        out_ref[0] = m

    @pl.when(i > 0)
    def _():
        out_ref[0] = jnp.maximum(out_ref[0], m)


def _head1(x1, x2, x3, wa, wb, wc, gam, bet, b, n, blk=512):
    nblk = n // blk
    xspec = pl.BlockSpec((blk, 64), lambda bb, ii: (bb * nblk + ii, 0))
    wspec = pl.BlockSpec((1024, 64), lambda bb, ii: (0, 0))
    vspec = pl.BlockSpec((1, 1024), lambda bb, ii: (0, 0))
    return pl.pallas_call(
        functools.partial(_head1_body, nblk=nblk),
        grid=(b, nblk),
        in_specs=[xspec, xspec, xspec, wspec, wspec, wspec, vspec, vspec],
        out_specs=pl.BlockSpec((1, 1, 1024), lambda bb, ii: (bb, 0, 0)),
        out_shape=jax.ShapeDtypeStruct((b, 1, 1024), jnp.float32),
    )(x1, x2, x3, wa, wb, wc, gam, bet)


def _head2_body(cat_ref, gmax_ref, x1_ref, x2_ref, x3_ref, emb_ref,
                w0g_ref, w0c_ref, w0x1_ref, w0x2_ref, w0x3_ref, g0_ref, b0_ref,
                w1_ref, g1_ref, b1_ref, w2_ref, b2_ref, out_ref):
    dn = (((1,), (1,)), ((), ()))

    def mm(a, b):
        return lax.dot_general(a, b, dn, preferred_element_type=jnp.float32)

    cid = cat_ref[0, 0, 0]
    cvec = emb_ref[pl.ds(cid, 1), :]
    const = mm(gmax_ref[0], w0g_ref[...]) + mm(cvec, w0c_ref[...])
    h = (mm(x1_ref[...], w0x1_ref[...]) + mm(x2_ref[...], w0x2_ref[...])
         + mm(x3_ref[...], w0x3_ref[...]) + const)
    h = _ln_gelu(h, g0_ref[...], b0_ref[...])
    h = _ln_gelu(mm(h, w1_ref[...]), g1_ref[...], b1_ref[...])
    out_ref[...] = mm(h, w2_ref[...]) + b2_ref[...]


def _head2(category, gmax, x1, x2, x3, emb, w0g, w0c, w0x1, w0x2, w0x3,
           g0, b0, w1, g1, b1, w2, b2, b, n, blk=512):
    nblk = n // blk
    xspec = pl.BlockSpec((blk, 64), lambda bb, ii: (bb * nblk + ii, 0))

    def fullspec(a):
        return pl.BlockSpec(a.shape, lambda bb, ii, nd=a.ndim: (0,) * nd)

    return pl.pallas_call(
        _head2_body,
        grid=(b, nblk),
        in_specs=[
            pl.BlockSpec((1, 1, 1), lambda bb, ii: (bb, 0, 0),
                         memory_space=pltpu.SMEM),
            pl.BlockSpec((1, 1, 1024), lambda bb, ii: (bb, 0, 0)),
            xspec, xspec, xspec,
            fullspec(emb), fullspec(w0g), fullspec(w0c), fullspec(w0x1),
            fullspec(w0x2), fullspec(w0x3), fullspec(g0), fullspec(b0),
            fullspec(w1), fullspec(g1), fullspec(b1), fullspec(w2),
            fullspec(b2),
        ],
        out_specs=pl.BlockSpec((blk, 50), lambda bb, ii: (bb * nblk + ii, 0)),
        out_shape=jax.ShapeDtypeStruct((b * n, 50), jnp.float32),
    )(category, gmax, x1, x2, x3, emb, w0g, w0c, w0x1, w0x2, w0x3,
      g0, b0, w1, g1, b1, w2, b2)


def kernel(x, xyz, category, b0_w0, b0_g0, b0_b0, b0_w1, b0_g1, b0_b1,
           b1_w0, b1_g0, b1_b0, b1_w1, b1_g1, b1_b1, b2_w0, b2_g0, b2_b0,
           mlp1_w, mlp1_g, mlp1_b, cat_emb, m2_w0, m2_g0, m2_b0,
           m2_w1, m2_g1, m2_b1, m2_w2, m2_bias2):
    b, n, _ = x.shape
    m = b * n

    def row(v):
        return v.reshape(1, -1).astype(jnp.float32)

    xyz8 = jnp.pad(xyz, ((0, 0), (0, 0), (0, 5)))
    idx1 = _knn(xyz8)
    x16 = jnp.pad(x, ((0, 0), (0, 0), (0, 13))).reshape(m, 16)
    w0p = (jnp.zeros((64, 32), jnp.float32)
           .at[:, 0:3].set(b0_w0[:, 0:3]).at[:, 16:19].set(b0_w0[:, 3:6]))
    g1 = _sc_gather(x16, idx1.transpose(1, 0, 2).reshape(-1))
    x1 = _edgeconv(g1.reshape(_K, m, 16), x16, w0p, row(b0_g0), row(b0_b0),
                   layer2=(b0_w1, row(b0_g1), row(b0_b1)))

    idx2 = _knn(x1.reshape(b, n, 64))
    g2 = _sc_gather(x1, idx2.transpose(1, 0, 2).reshape(-1))
    x2 = _edgeconv(g2.reshape(_K, m, 64), x1, b1_w0, row(b1_g0), row(b1_b0),
                   layer2=(b1_w1, row(b1_g1), row(b1_b1)))

    idx3 = _knn(x2.reshape(b, n, 64))
    y3, z3 = _pretrans(x2, b2_w0[:, 0:64], b2_w0[:, 64:128] - b2_w0[:, 0:64])
    g3 = _sc_gather(y3, idx3.transpose(1, 0, 2).reshape(-1))
    x3 = _edgeconv3(g3.reshape(_K, m, 64), z3, row(b2_g0), row(b2_b0))

    gmax = _head1(x1, x2, x3, mlp1_w[:, 0:64], mlp1_w[:, 64:128],
                  mlp1_w[:, 128:192], row(mlp1_g), row(mlp1_b), b, n)
    out = _head2(category.reshape(b, 1, 1).astype(jnp.int32), gmax, x1, x2, x3,
                 cat_emb, m2_w0[:, 0:1024], m2_w0[:, 1024:1088],
                 m2_w0[:, 1088:1152], m2_w0[:, 1152:1216], m2_w0[:, 1216:1280],
                 row(m2_g0), row(m2_b0), m2_w1, row(m2_g1), row(m2_b1),
                 m2_w2, row(m2_bias2), b, n)
    return out.reshape(b, n, 50)


# speedup vs baseline: 8.7718x; 1.0306x over previous
import functools

import jax
import jax.numpy as jnp
from jax import lax
from jax.experimental import pallas as pl
from jax.experimental.pallas import tpu as pltpu
from jax.experimental.pallas import tpu_sc as plsc

_K = 20


def _knn_body(f_ref, out_ref, *, n, k, blk):
    b = pl.program_id(0)
    i = pl.program_id(1)
    f = f_ref[0]
    rows = f_ref[0, pl.ds(i * blk, blk), :]
    prod = lax.dot_general(rows, f, (((1,), (1,)), ((), ())),
                           preferred_element_type=jnp.float32)
    rsq = jnp.sum(rows * rows, axis=1, keepdims=True)
    fsq = jnp.sum(f * f, axis=1)[None, :]
    d = (rsq + (-2.0 * prod)) + fsq
    iota = lax.broadcasted_iota(jnp.int32, (blk, n), 1)
    base = b * n
    inf = jnp.float32(jnp.inf)
    for j in range(k):
        m = jnp.min(d, axis=1, keepdims=True)
        ii = jnp.min(jnp.where(d == m, iota, n), axis=1)
        out_ref[0, j, :] = ii + base
        d = jnp.where(iota == ii[:, None], inf, d)


def _knn(feat, k=_K, blk=256):
    b, n, c = feat.shape
    return pl.pallas_call(
        functools.partial(_knn_body, n=n, k=k, blk=blk),
        grid=(b, n // blk),
        in_specs=[pl.BlockSpec((1, n, c), lambda bb, ii: (bb, 0, 0))],
        out_specs=pl.BlockSpec((1, k, blk), lambda bb, ii: (bb, 0, ii)),
        out_shape=jax.ShapeDtypeStruct((b, k, n), jnp.int32),
    )(feat)


_NC = 2
_NS = 16
_NW = _NC * _NS
_CH = 128


def _sc_gather(table, idx):
    e = idx.shape[0]
    d = table.shape[1]
    per_w = e // _NW
    nch = per_w // _CH
    idx3 = idx.reshape(_NW, nch, _CH)
    mesh = plsc.VectorSubcoreMesh(core_axis_name="c", subcore_axis_name="s")

    nbuf = 4
    scratch = [pltpu.VMEM((nch, _CH), jnp.int32)]
    scratch += [pltpu.VMEM((_CH, d), jnp.float32) for _ in range(nbuf)]
    scratch += [pltpu.SemaphoreType.DMA for _ in range(nbuf)]

    @functools.partial(
        pl.kernel,
        mesh=mesh,
        compiler_params=pltpu.CompilerParams(use_tc_tiling_on_sc=False),
        out_type=jax.ShapeDtypeStruct((e, d), jnp.float32),
        scratch_types=scratch,
    )
    def gk(table_hbm, idx_hbm, out_hbm, idx_v, *bufsem):
        bufs, sems = bufsem[:nbuf], bufsem[nbuf:]
        wid = lax.axis_index("s") * _NC + lax.axis_index("c")
        pltpu.sync_copy(idx_hbm.at[wid], idx_v)
        base = wid * per_w
        for p in range(nbuf - 1):
            pltpu.async_copy(table_hbm.at[idx_v.at[p]], bufs[p], sems[p])

        def step(g, s):
            pltpu.make_async_copy(table_hbm.at[idx_v.at[g]], bufs[s],
                                  sems[s]).wait()
            nx = (s + nbuf - 1) % nbuf

            @pl.when(g + nbuf - 1 < nch)
            def _():
                pltpu.async_copy(table_hbm.at[idx_v.at[g + nbuf - 1]],
                                 bufs[nx], sems[nx])

            pltpu.sync_copy(bufs[s], out_hbm.at[pl.ds(base + g * _CH, _CH)])

        def rnd(h, carry):
            for s in range(nbuf):
                step(nbuf * h + s, s)
            return carry

        lax.fori_loop(0, nch // nbuf, rnd, 0)

    return gk(table, idx3)


def _ln_gelu(h, gam, bet):
    mu = jnp.mean(h, axis=1, keepdims=True)
    v = jnp.mean((h - mu) ** 2, axis=1, keepdims=True)
    h = (h - mu) / jnp.sqrt(v + 1e-5) * gam + bet
    return h * 0.5 * (1.0 + lax.erf(h / jnp.sqrt(jnp.float32(2.0))))


def _pre_body(x_ref, wa_ref, wd_ref, y_ref, z_ref):
    dn = (((1,), (1,)), ((), ()))
    x = x_ref[...]
    y_ref[...] = lax.dot_general(x, wa_ref[...], dn,
                                 preferred_element_type=jnp.float32)
    z_ref[...] = lax.dot_general(x, wd_ref[...], dn,
                                 preferred_element_type=jnp.float32)


def _pretrans(x, wa, wd, blk=1024):
    m, d = x.shape
    ospec = pl.BlockSpec((blk, 64), lambda i: (i, 0))
    oshape = jax.ShapeDtypeStruct((m, 64), jnp.float32)
    return pl.pallas_call(
        _pre_body,
        grid=(m // blk,),
        in_specs=[
            pl.BlockSpec((blk, d), lambda i: (i, 0)),
            pl.BlockSpec(wa.shape, lambda i: (0, 0)),
            pl.BlockSpec(wd.shape, lambda i: (0, 0)),
        ],
        out_specs=[ospec, ospec],
        out_shape=[oshape, oshape],
    )(x, wa, wd)


def _edge_body(*refs, k, two, pre):
    gy_ref, xc_ref, w0_ref, g0_ref, b0_ref = refs[:5]
    i = 8 if two else 5
    dn = (((1,), (1,)), ((), ()))
    xc = xc_ref[...]
    w0 = w0_ref[...]
    g0, b0 = g0_ref[...], b0_ref[...]
    if two:
        w1, g1, b1 = refs[5][...], refs[6][...], refs[7][...]
    if pre:
        wa_ref, wd_ref = refs[i], refs[i + 1]
        i += 2
    out_ref = refs[i]
    acc = None
    for j in range(k):
        h = jnp.concatenate([gy_ref[j] - xc, xc], axis=1)
        h = lax.dot_general(h, w0, dn, preferred_element_type=jnp.float32)
        h = _ln_gelu(h, g0, b0)
        if two:
            h = lax.dot_general(h, w1, dn, preferred_element_type=jnp.float32)
            h = _ln_gelu(h, g1, b1)
        acc = h if acc is None else jnp.maximum(acc, h)
    out_ref[...] = acc
    if pre:
        refs[i + 1][...] = lax.dot_general(acc, wa_ref[...], dn,
                                           preferred_element_type=jnp.float32)
        refs[i + 2][...] = lax.dot_general(acc, wd_ref[...], dn,
                                           preferred_element_type=jnp.float32)


def _edge3_body(gy_ref, z_ref, g0_ref, b0_ref, out_ref, *, k):
    z = z_ref[...]
    g0, b0 = g0_ref[...], b0_ref[...]
    acc = None
    for j in range(k):
        h = _ln_gelu(gy_ref[j] + z, g0, b0)
        acc = h if acc is None else jnp.maximum(acc, h)
    out_ref[...] = acc


def _edgeconv3(gy, z, g0, b0, k=_K, blk=1024):
    m = z.shape[0]
    return pl.pallas_call(
        functools.partial(_edge3_body, k=k),
        grid=(m // blk,),
        in_specs=[
            pl.BlockSpec((k, blk, 64), lambda i: (0, i, 0)),
            pl.BlockSpec((blk, 64), lambda i: (i, 0)),
            pl.BlockSpec(g0.shape, lambda i: (0, 0)),
            pl.BlockSpec(b0.shape, lambda i: (0, 0)),
        ],
        out_specs=pl.BlockSpec((blk, 64), lambda i: (i, 0)),
        out_shape=jax.ShapeDtypeStruct((m, 64), jnp.float32),
    )(gy, z, g0, b0)


def _edgeconv(gy, xc, w0, g0, b0, layer2=None, pre=None, k=_K, blk=1024):
    m, d = xc.shape
    extra = (list(layer2) if layer2 is not None else []) + \
            (list(pre) if pre is not None else [])
    in_specs = [
        pl.BlockSpec((k, blk, d), lambda i: (0, i, 0)),
        pl.BlockSpec((blk, d), lambda i: (i, 0)),
        pl.BlockSpec(w0.shape, lambda i: (0, 0)),
        pl.BlockSpec(g0.shape, lambda i: (0, 0)),
        pl.BlockSpec(b0.shape, lambda i: (0, 0)),
    ] + [pl.BlockSpec(a.shape, lambda i, nd=a.ndim: (0,) * nd) for a in extra]
    ospec = pl.BlockSpec((blk, 64), lambda i: (i, 0))
    oshape = jax.ShapeDtypeStruct((m, 64), jnp.float32)
    nout = 3 if pre is not None else 1
    return pl.pallas_call(
        functools.partial(_edge_body, k=k, two=layer2 is not None,
                          pre=pre is not None),
        grid=(m // blk,),
        in_specs=in_specs,
        out_specs=[ospec] * nout if pre is not None else ospec,
        out_shape=[oshape] * nout if pre is not None else oshape,
    )(gy, xc, w0, g0, b0, *extra)


def _head1_body(x1_ref, x2_ref, x3_ref, wa_ref, wb_ref, wc_ref, gam_ref,
                bet_ref, out_ref, *, nblk):
    i = pl.program_id(1)
    dn = (((1,), (1,)), ((), ()))
    h = (lax.dot_general(x1_ref[...], wa_ref[...], dn,
                         preferred_element_type=jnp.float32)
         + lax.dot_general(x2_ref[...], wb_ref[...], dn,
                           preferred_element_type=jnp.float32)
         + lax.dot_general(x3_ref[...], wc_ref[...], dn,
                           preferred_element_type=jnp.float32))
    h = _ln_gelu(h, gam_ref[...], bet_ref[...])
    m = jnp.max(h, axis=0, keepdims=True)

    @pl.when(i == 0)
    def _():
        out_ref[0] = m

    @pl.when(i > 0)
    def _():
        out_ref[0] = jnp.maximum(out_ref[0], m)


def _head1(x1, x2, x3, wa, wb, wc, gam, bet, b, n, blk=512):
    nblk = n // blk
    xspec = pl.BlockSpec((blk, 64), lambda bb, ii: (bb * nblk + ii, 0))
    wspec = pl.BlockSpec((1024, 64), lambda bb, ii: (0, 0))
    vspec = pl.BlockSpec((1, 1024), lambda bb, ii: (0, 0))
    return pl.pallas_call(
        functools.partial(_head1_body, nblk=nblk),
        grid=(b, nblk),
        in_specs=[xspec, xspec, xspec, wspec, wspec, wspec, vspec, vspec],
        out_specs=pl.BlockSpec((1, 1, 1024), lambda bb, ii: (bb, 0, 0)),
        out_shape=jax.ShapeDtypeStruct((b, 1, 1024), jnp.float32),
    )(x1, x2, x3, wa, wb, wc, gam, bet)


def _head2_body(cat_ref, gmax_ref, x1_ref, x2_ref, x3_ref, emb_ref,
                w0g_ref, w0c_ref, w0x1_ref, w0x2_ref, w0x3_ref, g0_ref, b0_ref,
                w1_ref, g1_ref, b1_ref, w2_ref, b2_ref, out_ref):
    dn = (((1,), (1,)), ((), ()))

    def mm(a, b):
        return lax.dot_general(a, b, dn, preferred_element_type=jnp.float32)

    cid = cat_ref[0, 0, 0]
    cvec = emb_ref[pl.ds(cid, 1), :]
    const = mm(gmax_ref[0], w0g_ref[...]) + mm(cvec, w0c_ref[...])
    h = (mm(x1_ref[...], w0x1_ref[...]) + mm(x2_ref[...], w0x2_ref[...])
         + mm(x3_ref[...], w0x3_ref[...]) + const)
    h = _ln_gelu(h, g0_ref[...], b0_ref[...])
    h = _ln_gelu(mm(h, w1_ref[...]), g1_ref[...], b1_ref[...])
    out_ref[...] = mm(h, w2_ref[...]) + b2_ref[...]


def _head2(category, gmax, x1, x2, x3, emb, w0g, w0c, w0x1, w0x2, w0x3,
           g0, b0, w1, g1, b1, w2, b2, b, n, blk=512):
    nblk = n // blk
    xspec = pl.BlockSpec((blk, 64), lambda bb, ii: (bb * nblk + ii, 0))

    def fullspec(a):
        return pl.BlockSpec(a.shape, lambda bb, ii, nd=a.ndim: (0,) * nd)

    return pl.pallas_call(
        _head2_body,
        grid=(b, nblk),
        in_specs=[
            pl.BlockSpec((1, 1, 1), lambda bb, ii: (bb, 0, 0),
                         memory_space=pltpu.SMEM),
            pl.BlockSpec((1, 1, 1024), lambda bb, ii: (bb, 0, 0)),
            xspec, xspec, xspec,
            fullspec(emb), fullspec(w0g), fullspec(w0c), fullspec(w0x1),
            fullspec(w0x2), fullspec(w0x3), fullspec(g0), fullspec(b0),
            fullspec(w1), fullspec(g1), fullspec(b1), fullspec(w2),
            fullspec(b2),
        ],
        out_specs=pl.BlockSpec((blk, 50), lambda bb, ii: (bb * nblk + ii, 0)),
        out_shape=jax.ShapeDtypeStruct((b * n, 50), jnp.float32),
    )(category, gmax, x1, x2, x3, emb, w0g, w0c, w0x1, w0x2, w0x3,
      g0, b0, w1, g1, b1, w2, b2)


def kernel(x, xyz, category, b0_w0, b0_g0, b0_b0, b0_w1, b0_g1, b0_b1,
           b1_w0, b1_g0, b1_b0, b1_w1, b1_g1, b1_b1, b2_w0, b2_g0, b2_b0,
           mlp1_w, mlp1_g, mlp1_b, cat_emb, m2_w0, m2_g0, m2_b0,
           m2_w1, m2_g1, m2_b1, m2_w2, m2_bias2):
    b, n, _ = x.shape
    m = b * n

    def row(v):
        return v.reshape(1, -1).astype(jnp.float32)

    xyz8 = jnp.pad(xyz, ((0, 0), (0, 0), (0, 5)))
    idx1 = _knn(xyz8)
    x16 = jnp.pad(x, ((0, 0), (0, 0), (0, 13))).reshape(m, 16)
    w0p = (jnp.zeros((64, 32), jnp.float32)
           .at[:, 0:3].set(b0_w0[:, 0:3]).at[:, 16:19].set(b0_w0[:, 3:6]))
    g1 = _sc_gather(x16, idx1.transpose(1, 0, 2).reshape(-1))
    x1 = _edgeconv(g1.reshape(_K, m, 16), x16, w0p, row(b0_g0), row(b0_b0),
                   layer2=(b0_w1, row(b0_g1), row(b0_b1)))

    idx2 = _knn(x1.reshape(b, n, 64))
    g2 = _sc_gather(x1, idx2.transpose(1, 0, 2).reshape(-1))
    x2, y3, z3 = _edgeconv(g2.reshape(_K, m, 64), x1, b1_w0,
                           row(b1_g0), row(b1_b0),
                           layer2=(b1_w1, row(b1_g1), row(b1_b1)),
                           pre=(b2_w0[:, 0:64],
                                b2_w0[:, 64:128] - b2_w0[:, 0:64]))

    idx3 = _knn(x2.reshape(b, n, 64))
    g3 = _sc_gather(y3, idx3.transpose(1, 0, 2).reshape(-1))
    x3 = _edgeconv3(g3.reshape(_K, m, 64), z3, row(b2_g0), row(b2_b0))

    gmax = _head1(x1, x2, x3, mlp1_w[:, 0:64], mlp1_w[:, 64:128],
                  mlp1_w[:, 128:192], row(mlp1_g), row(mlp1_b), b, n)
    out = _head2(category.reshape(b, 1, 1).astype(jnp.int32), gmax, x1, x2, x3,
                 cat_emb, m2_w0[:, 0:1024], m2_w0[:, 1024:1088],
                 m2_w0[:, 1088:1152], m2_w0[:, 1152:1216], m2_w0[:, 1216:1280],
                 row(m2_g0), row(m2_b0), m2_w1, row(m2_g1), row(m2_b1),
                 m2_w2, row(m2_bias2), b, n)
    return out.reshape(b, n, 50)
